# Initial kernel scaffold; baseline (speedup 1.0000x reference)
import jax
import jax.numpy as jnp
from jax.experimental import pallas as pl


def kernel(x, paths, b):
    raise NotImplementedError("write your pallas kernel here")


# speedup vs baseline: 4.2081x; 4.2081x over previous
import jax
import jax.numpy as jnp
from jax.experimental import pallas as pl
from jax.experimental.pallas import tpu as pltpu

BATCH = 64
BLOCK = 72
MAX_PATH = 5
NUM_NODES = BATCH * BLOCK


def _spatial_kernel(b_ref, paths_ref, out_ref):
    i = pl.program_id(0)
    p = paths_ref[0]
    lengths = jnp.sum((p != -1).astype(jnp.int32), axis=0)
    vals = jnp.zeros((BLOCK, BLOCK), dtype=jnp.float32)
    for k in range(MAX_PATH + 1):
        vals = jnp.where(lengths == k, b_ref[k], vals)
    tiled = pltpu.repeat(vals, BATCH, axis=1)
    col = jax.lax.broadcasted_iota(jnp.int32, (BLOCK, NUM_NODES), 1)
    mask = (col // BLOCK) == i
    out_ref[...] = jnp.where(mask, tiled, 0.0)


def kernel(x, paths, b):
    del x
    p32 = jnp.transpose(paths.astype(jnp.int32), (0, 3, 1, 2))
    return pl.pallas_call(
        _spatial_kernel,
        grid=(BATCH,),
        in_specs=[
            pl.BlockSpec(memory_space=pltpu.SMEM),
            pl.BlockSpec((1, MAX_PATH, BLOCK, BLOCK), lambda i: (i, 0, 0, 0)),
        ],
        out_specs=pl.BlockSpec((BLOCK, NUM_NODES), lambda i: (i, 0)),
        out_shape=jax.ShapeDtypeStruct((NUM_NODES, NUM_NODES), jnp.float32),
    )(b, p32)
